# baseline (speedup 1.0000x reference)
import math

import jax
import jax.numpy as jnp
from jax.experimental import pallas as pl
from jax.experimental.pallas import tpu as pltpu

_S = 2048
_HD = 128
_DL = 16
_TOPK = 8
_TILE = 256
_NT = _S // _TILE
_SC2 = 1.0 / math.sqrt(_DL)
_NEG_INF = float("-inf")


def _dot_t(a, b):
    return jax.lax.dot_general(
        a, b, (((1,), (1,)), ((), ())), preferred_element_type=jnp.float32
    )


def _attn_kernel(q_ref, k_ref, vl_ref, vp_ref,
                 wql_ref, bql_ref, wkl_ref, bkl_ref,
                 wqh_ref, bqh_ref, wkh_ref, bkh_ref,
                 out_ref, s_ref, aux_ref):
    i = pl.program_id(0)
    nsteps = pl.num_programs(0)
    par = jax.lax.rem(i, 2)

    def body(do_cur, do_prev):
        if do_cur:
            q = q_ref[0]
            k = k_ref[0]
            vl = vl_ref[0]
            q_low = (_dot_t(q, wql_ref[...]) + bql_ref[...]) * _SC2
            k_low = _dot_t(k, wkl_ref[...]) + bkl_ref[...]
            aux_ref[par, :, 0:_DL] = (_dot_t(q, wqh_ref[...])
                                      + bqh_ref[...]) * _SC2
            aux_ref[par, :, _DL:2 * _DL] = _dot_t(k, wkh_ref[...]) + bkh_ref[...]
            rows = jax.lax.broadcasted_iota(jnp.int32, (_S, _TILE), 0)
            t1 = jnp.full((_S, 1), _NEG_INF, dtype=jnp.float32)

        if do_prev:
            buf = 1 - par
            q_high = aux_ref[buf, :, 0:_DL]
            k_high = aux_ref[buf, :, _DL:2 * _DL]
            thr = aux_ref[buf, :, 2 * _DL:2 * _DL + 1]

        def s1_tile(t):
            lo = t * _TILE
            s_tile = _dot_t(q_low, k_low[lo:lo + _TILE, :])
            cond = rows == vl[:, lo:lo + _TILE]
            s_tile = jnp.where(cond, _NEG_INF, s_tile)
            s_ref[par, :, lo:lo + _TILE] = s_tile
            return jnp.max(s_tile, axis=1, keepdims=True)

        if do_cur:
            for t in range(_NT):
                t1 = jnp.maximum(t1, s1_tile(t))
            aux_ref[par, :, 2 * _DL:2 * _DL + 1] = t1

        for p in range(_TOPK - 1):
            if do_prev:
                m = jnp.full((_S, 1), _NEG_INF, dtype=jnp.float32)
                for t in range(_NT):
                    lo = t * _TILE
                    tile = s_ref[buf, :, lo:lo + _TILE]
                    cand = jnp.where(tile < thr, tile, _NEG_INF)
                    m = jnp.maximum(m, jnp.max(cand, axis=1, keepdims=True))
                thr = m

        if do_prev:
            acc = jnp.zeros((_S, _HD), dtype=jnp.float32)
            for t in range(_NT):
                lo = t * _TILE
                tile = s_ref[buf, :, lo:lo + _TILE]
                sh = _dot_t(q_high, k_high[lo:lo + _TILE, :])
                corr = jnp.where(tile >= thr, sh, tile)
                cmax = jnp.max(corr, axis=0, keepdims=True)
                e = jnp.exp(corr - cmax)
                csum = jnp.sum(e, axis=0, keepdims=True)
                attn = e / csum
                acc = acc + jnp.dot(attn, vp_ref[0][lo:lo + _TILE, :],
                                    preferred_element_type=jnp.float32)
            out_ref[0] = acc

    @pl.when(jnp.logical_and(i > 0, i < nsteps - 1))
    def _():
        body(True, True)

    @pl.when(i == 0)
    def _():
        body(True, False)

    @pl.when(i == nsteps - 1)
    def _():
        body(False, True)


def kernel(queries, keys, values, valid_lens, Wq_low, bq_low, Wk_low, bk_low,
           Wq_high, bq_high, Wk_high, bk_high):
    b, s, hd = queries.shape
    vl3 = jnp.clip(valid_lens, 0, s - 1).reshape(b, 1, s)
    bql = bq_low.reshape(1, _DL)
    bkl = bk_low.reshape(1, _DL)
    bqh = bq_high.reshape(1, _DL)
    bkh = bk_high.reshape(1, _DL)

    last = b - 1
    cur = lambda i: (jnp.minimum(i, last), 0, 0)
    prev = lambda i: (jnp.maximum(i - 1, 0), 0, 0)
    full = lambda shape: pl.BlockSpec(shape, lambda i: (0,) * len(shape))

    return pl.pallas_call(
        _attn_kernel,
        grid=(b + 1,),
        in_specs=[
            pl.BlockSpec((1, _S, _HD), cur),
            pl.BlockSpec((1, _S, _HD), cur),
            pl.BlockSpec((1, 1, _S), cur),
            pl.BlockSpec((1, _S, _HD), prev),
            full((_DL, _HD)), full((1, _DL)),
            full((_DL, _HD)), full((1, _DL)),
            full((_DL, _HD)), full((1, _DL)),
            full((_DL, _HD)), full((1, _DL)),
        ],
        out_specs=pl.BlockSpec((1, _S, _HD), prev),
        out_shape=jax.ShapeDtypeStruct((b, _S, _HD), jnp.float32),
        scratch_shapes=[pltpu.VMEM((2, _S, _S), jnp.float32),
                        pltpu.VMEM((2, _S, _HD), jnp.float32)],
    )(queries, keys, vl3, values, Wq_low, bql, Wk_low, bkl,
      Wq_high, bqh, Wk_high, bkh)


# speedup vs baseline: 2.5020x; 2.5020x over previous
import math

import jax
import jax.numpy as jnp
from jax.experimental import pallas as pl
from jax.experimental.pallas import tpu as pltpu

_S = 2048
_HD = 128
_DL = 16
_TOPK = 8
_TILE = 512
_NT = _S // _TILE
_SC2 = 1.0 / math.sqrt(_DL)
_NEG_INF = float("-inf")


def _dot_t(a, b):
    return jax.lax.dot_general(
        a, b, (((1,), (1,)), ((), ())), preferred_element_type=jnp.float32
    )


def _attn_kernel(q_ref, k_ref, v_ref, vl_ref,
                 wql_ref, bql_ref, wkl_ref, bkl_ref,
                 wqh_ref, bqh_ref, wkh_ref, bkh_ref,
                 out_ref, s_ref):
    q = q_ref[0]
    k = k_ref[0]
    vl = vl_ref[0]

    q_low = (_dot_t(q, wql_ref[...]) + bql_ref[...]) * _SC2
    k_low = _dot_t(k, wkl_ref[...]) + bkl_ref[...]
    q_high = (_dot_t(q, wqh_ref[...]) + bqh_ref[...]) * _SC2
    k_high = _dot_t(k, wkh_ref[...]) + bkh_ref[...]

    rows = jax.lax.broadcasted_iota(jnp.int32, (_S, _TILE), 0)
    thr = jnp.full((_S, 1), _NEG_INF, dtype=jnp.float32)
    for t in range(_NT):
        lo = t * _TILE
        s_tile = _dot_t(q_low, k_low[lo:lo + _TILE, :])
        cond = rows == vl[:, lo:lo + _TILE]
        s_tile = jnp.where(cond, _NEG_INF, s_tile)
        s_ref[:, lo:lo + _TILE] = s_tile
        thr = jnp.maximum(thr, jnp.max(s_tile, axis=1, keepdims=True))

    for p in range(_TOPK - 1):
        m = jnp.full((_S, 1), _NEG_INF, dtype=jnp.float32)
        for t in range(_NT):
            lo = t * _TILE
            tile = s_ref[:, lo:lo + _TILE]
            cand = jnp.where(tile < thr, tile, _NEG_INF)
            m = jnp.maximum(m, jnp.max(cand, axis=1, keepdims=True))
        thr = m

    acc = jnp.zeros((_S, _HD), dtype=jnp.float32)
    for t in range(_NT):
        lo = t * _TILE
        tile = s_ref[:, lo:lo + _TILE]
        sh = _dot_t(q_high, k_high[lo:lo + _TILE, :])
        corr = jnp.where(tile >= thr, sh, tile)
        cmax = jnp.max(corr, axis=0, keepdims=True)
        e = jnp.exp(corr - cmax)
        csum = jnp.sum(e, axis=0, keepdims=True)
        attn = e / csum
        acc = acc + jnp.dot(attn, v_ref[0][lo:lo + _TILE, :],
                            preferred_element_type=jnp.float32)
    out_ref[0] = acc


def kernel(queries, keys, values, valid_lens, Wq_low, bq_low, Wk_low, bk_low,
           Wq_high, bq_high, Wk_high, bk_high):
    b, s, hd = queries.shape
    vl3 = jnp.clip(valid_lens, 0, s - 1).reshape(b, 1, s)
    bql = bq_low.reshape(1, _DL)
    bkl = bk_low.reshape(1, _DL)
    bqh = bq_high.reshape(1, _DL)
    bkh = bk_high.reshape(1, _DL)

    full = lambda shape: pl.BlockSpec(shape, lambda i: (0,) * len(shape))
    per_b = lambda shape: pl.BlockSpec(shape, lambda i: (i,) + (0,) * (len(shape) - 1))

    return pl.pallas_call(
        _attn_kernel,
        grid=(b,),
        in_specs=[
            per_b((1, _S, _HD)),
            per_b((1, _S, _HD)),
            per_b((1, _S, _HD)),
            per_b((1, 1, _S)),
            full((_DL, _HD)), full((1, _DL)),
            full((_DL, _HD)), full((1, _DL)),
            full((_DL, _HD)), full((1, _DL)),
            full((_DL, _HD)), full((1, _DL)),
        ],
        out_specs=per_b((1, _S, _HD)),
        out_shape=jax.ShapeDtypeStruct((b, _S, _HD), jnp.float32),
        scratch_shapes=[pltpu.VMEM((_S, _S), jnp.float32)],
    )(queries, keys, values, vl3, Wq_low, bql, Wk_low, bkl,
      Wq_high, bqh, Wk_high, bkh)
